# Initial kernel scaffold; baseline (speedup 1.0000x reference)
import jax
import jax.numpy as jnp
from jax.experimental import pallas as pl


def kernel(X, edge_index, edge_type, bases0, comb0, bases1, comb1):
    raise NotImplementedError("write your pallas kernel here")


# speedup vs baseline: 14.0695x; 14.0695x over previous
import functools

import jax
import jax.numpy as jnp
from jax import lax
from jax.experimental import pallas as pl
from jax.experimental.pallas import tpu as pltpu
from jax.experimental.pallas import tpu_sc as plsc

_NC = 2
_NS = 16
_LN = 16
_NW = _NC * _NS
_DW = 16


def _tc_stage(n, d, r, b, bn, input_mode, emit_xw):
  grid = (n // bn,)

  def body(*refs):
    if input_mode == "x":
      x_ref, rest = refs[0], refs[1:]
      h = x_ref[...]
    else:
      acc_ref, deg_ref, rest = refs[0], refs[1], refs[2:]
      a = acc_ref[0] + acc_ref[1]
      dg = deg_ref[0] + deg_ref[1]
      h = a / jnp.maximum(dg, 1.0)
    if not emit_xw:
      out_ref = rest[0]
      out_ref[...] = h
      return
    bases_ref, comb_ref, out_ref = rest
    for ri in range(r):
      w = comb_ref[ri, 0] * bases_ref[0]
      for bi in range(1, b):
        w = w + comb_ref[ri, bi] * bases_ref[bi]
      out_ref[ri] = jnp.dot(h, w, preferred_element_type=jnp.float32)

  in_specs = []
  if input_mode == "x":
    in_specs.append(pl.BlockSpec((bn, d), lambda i: (i, 0)))
  else:
    in_specs.append(pl.BlockSpec((2, bn, d), lambda i: (0, i, 0)))
    in_specs.append(pl.BlockSpec((2, bn, d), lambda i: (0, i, 0)))
  if emit_xw:
    in_specs.append(pl.BlockSpec((b, d, d), lambda i: (0, 0, 0)))
    in_specs.append(pl.BlockSpec(memory_space=pltpu.SMEM))
    out_spec = pl.BlockSpec((r, bn, d), lambda i: (0, i, 0))
    out_shape = jax.ShapeDtypeStruct((r, n, d), jnp.float32)
  else:
    out_spec = pl.BlockSpec((bn, d), lambda i: (i, 0))
    out_shape = jax.ShapeDtypeStruct((n, d), jnp.float32)

  return pl.pallas_call(
      body, grid=grid, in_specs=in_specs, out_specs=out_spec,
      out_shape=out_shape)


def _edge_index_stage(e2, n):
  def body(et_ref, src_ref, out_ref):
    out_ref[...] = et_ref[...] * n + src_ref[...]

  return pl.pallas_call(
      body,
      out_shape=jax.ShapeDtypeStruct((e2, 128), jnp.int32))


def _sc_pass(np_, e, d, chunk):
  ep = e // _NW
  nchunk = ep // chunk
  nrc = np_ // chunk
  jmax = nrc // _NS
  assert ep % chunk == 0 and chunk % _LN == 0 and chunk % 8 == 0
  assert chunk <= 128 and np_ % (chunk * _NS) == 0

  mesh = plsc.VectorSubcoreMesh(core_axis_name="c", subcore_axis_name="s")

  @functools.partial(
      pl.kernel, mesh=mesh,
      out_type=jax.ShapeDtypeStruct((_NC * np_, d), jnp.float32),
      scratch_types=[
          pltpu.VMEM((chunk,), jnp.int32),
          pltpu.VMEM((chunk,), jnp.int32),
          pltpu.VMEM((chunk, d), jnp.float32),
          pltpu.VMEM_SHARED((np_, d), jnp.float32),
          pltpu.SemaphoreType.DMA,
      ])
  def body(xw, idxs, dsts, zfeat, acc_out, idx_v, dst_v, rows_v, acc_s, sem):
    c = lax.axis_index("c")
    s = lax.axis_index("s")
    wid = s * _NC + c

    for j in range(jmax):
      cid = s * jmax + j
      pltpu.sync_copy(zfeat, acc_s.at[pl.ds(cid * chunk, chunk)])
    plsc.subcore_barrier()

    ebase = wid * ep

    def edge_body(i, _):
      off = ebase + i * chunk
      pltpu.sync_copy(idxs.at[pl.ds(off, chunk)], idx_v)
      pltpu.sync_copy(dsts.at[pl.ds(off, chunk)], dst_v)
      pltpu.async_copy(xw.at[idx_v], rows_v, sem).wait()
      pltpu.sync_copy(rows_v, acc_s.at[dst_v], add=True)
      return 0
    lax.fori_loop(0, nchunk, edge_body, 0)

    plsc.subcore_barrier()

    for j in range(jmax):
      cid = s * jmax + j
      pltpu.sync_copy(acc_s.at[pl.ds(cid * chunk, chunk)],
                      acc_out.at[pl.ds(c * np_ + cid * chunk, chunk)])

  return body


def _sc_deg_pass(np_, e, d, chunk):
  ep = e // _NW
  nchunk = ep // chunk
  jmax = np_ // chunk // _NS

  mesh = plsc.VectorSubcoreMesh(core_axis_name="c", subcore_axis_name="s")

  @functools.partial(
      pl.kernel, mesh=mesh,
      out_type=jax.ShapeDtypeStruct((_NC * np_, d), jnp.float32),
      scratch_types=[
          pltpu.VMEM((chunk,), jnp.int32),
          pltpu.VMEM((chunk, d), jnp.float32),
          pltpu.VMEM_SHARED((np_, d), jnp.float32),
      ])
  def body(dsts, zfeat, ones, deg_out, dst_v, ones_v, deg_s):
    c = lax.axis_index("c")
    s = lax.axis_index("s")
    wid = s * _NC + c

    for j in range(jmax):
      cid = s * jmax + j
      pltpu.sync_copy(zfeat, deg_s.at[pl.ds(cid * chunk, chunk)])
    pltpu.sync_copy(ones, ones_v)
    plsc.subcore_barrier()

    ebase = wid * ep

    def edge_body(i, _):
      off = ebase + i * chunk
      pltpu.sync_copy(dsts.at[pl.ds(off, chunk)], dst_v)
      pltpu.sync_copy(ones_v, deg_s.at[dst_v], add=True)
      return 0
    lax.fori_loop(0, nchunk, edge_body, 0)

    plsc.subcore_barrier()

    for j in range(jmax):
      cid = s * jmax + j
      pltpu.sync_copy(deg_s.at[pl.ds(cid * chunk, chunk)],
                      deg_out.at[pl.ds(c * np_ + cid * chunk, chunk)])

  return body


def kernel(X, edge_index, edge_type, bases0, comb0, bases1, comb1):
  n, d = X.shape
  e = edge_type.shape[0]
  b = bases0.shape[0]
  r = comb0.shape[0]
  bn = 1000
  chunk = 80
  np_ = -(-n // (chunk * _NS)) * (chunk * _NS)

  src = edge_index[0]
  dst = edge_index[1]
  e2 = e // 128
  flat_idx = _edge_index_stage(e2, n)(
      edge_type.reshape(e2, 128), src.reshape(e2, 128)).reshape(e)

  zfeat = jnp.zeros((chunk, d), jnp.float32)
  ones = jnp.ones((chunk, d), jnp.float32)

  def take(parts):
    return parts.reshape(_NC, np_, d)[:, :n]

  sc_pass = _sc_pass(np_, e, d, chunk)
  deg = take(_sc_deg_pass(np_, e, d, chunk)(dst, zfeat, ones))

  xw0 = _tc_stage(n, d, r, b, bn, "x", True)(X, bases0, comb0)
  acc0 = take(sc_pass(xw0.reshape(r * n, d), flat_idx, dst, zfeat))

  xw1 = _tc_stage(n, d, r, b, bn, "norm", True)(acc0, deg, bases1, comb1)
  acc1 = take(sc_pass(xw1.reshape(r * n, d), flat_idx, dst, zfeat))

  return _tc_stage(n, d, r, b, bn, "norm", False)(acc1, deg)


# speedup vs baseline: 28.3203x; 2.0129x over previous
import functools

import jax
import jax.numpy as jnp
from jax import lax
from jax.experimental import pallas as pl
from jax.experimental.pallas import tpu as pltpu
from jax.experimental.pallas import tpu_sc as plsc

_NC = 2
_NS = 16
_LN = 16
_NW = _NC * _NS
_DW = 16


def _tc_stage(n, d, r, b, bn, input_mode, emit_xw):
  grid = (n // bn,)

  def body(*refs):
    if input_mode == "x":
      x_ref, rest = refs[0], refs[1:]
      h = x_ref[...]
    else:
      acc_ref, deg_ref, rest = refs[0], refs[1], refs[2:]
      a = acc_ref[0] + acc_ref[1]
      dg = deg_ref[0] + deg_ref[1]
      h = a / jnp.maximum(dg, 1.0)
    if not emit_xw:
      out_ref = rest[0]
      out_ref[...] = h
      return
    bases_ref, comb_ref, out_ref = rest
    for ri in range(r):
      w = comb_ref[ri, 0] * bases_ref[0]
      for bi in range(1, b):
        w = w + comb_ref[ri, bi] * bases_ref[bi]
      out_ref[ri] = jnp.dot(h, w, preferred_element_type=jnp.float32)

  in_specs = []
  if input_mode == "x":
    in_specs.append(pl.BlockSpec((bn, d), lambda i: (i, 0)))
  else:
    in_specs.append(pl.BlockSpec((2, bn, d), lambda i: (0, i, 0)))
    in_specs.append(pl.BlockSpec((2, bn, d), lambda i: (0, i, 0)))
  if emit_xw:
    in_specs.append(pl.BlockSpec((b, d, d), lambda i: (0, 0, 0)))
    in_specs.append(pl.BlockSpec(memory_space=pltpu.SMEM))
    out_spec = pl.BlockSpec((r, bn, d), lambda i: (0, i, 0))
    out_shape = jax.ShapeDtypeStruct((r, n, d), jnp.float32)
  else:
    out_spec = pl.BlockSpec((bn, d), lambda i: (i, 0))
    out_shape = jax.ShapeDtypeStruct((n, d), jnp.float32)

  return pl.pallas_call(
      body, grid=grid, in_specs=in_specs, out_specs=out_spec,
      out_shape=out_shape)


def _edge_index_stage(e2, n):
  def body(et_ref, src_ref, out_ref):
    out_ref[...] = et_ref[...] * n + src_ref[...]

  return pl.pallas_call(
      body,
      out_shape=jax.ShapeDtypeStruct((e2, 128), jnp.int32))


def _sc_pass(np_, e, d, nch):
  ck = 128
  rc = 80
  jmax = np_ // rc // _NS
  assert e == nch * ck * _NW and np_ % (rc * _NS) == 0 and nch % 2 == 0

  mesh = plsc.VectorSubcoreMesh(core_axis_name="c", subcore_axis_name="s")

  @functools.partial(
      pl.kernel, mesh=mesh,
      out_type=jax.ShapeDtypeStruct((_NC * np_, d), jnp.float32),
      scratch_types=[
          pltpu.VMEM((nch * ck,), jnp.int32),
          pltpu.VMEM((ck,), jnp.int32),
          pltpu.VMEM((ck,), jnp.int32),
          pltpu.VMEM((ck, d), jnp.float32),
          pltpu.VMEM((ck, d), jnp.float32),
          pltpu.VMEM_SHARED((np_, d), jnp.float32),
          pltpu.SemaphoreType.DMA,
          pltpu.SemaphoreType.DMA,
          pltpu.SemaphoreType.DMA,
          pltpu.SemaphoreType.DMA,
      ])
  def body(xw, gi, dp, zfeat, acc_out, gi_v, dst0, dst1, rows0, rows1,
           acc_s, g0, g1, l0, l1):
    c = lax.axis_index("c")
    s = lax.axis_index("s")
    wid = s * _NC + c
    dsts = (dst0, dst1)
    rows = (rows0, rows1)
    gsem = (g0, g1)
    lsem = (l0, l1)

    for j in range(jmax):
      cid = s * jmax + j
      pltpu.sync_copy(zfeat, acc_s.at[pl.ds(cid * rc, rc)])
    pltpu.sync_copy(gi.at[wid], gi_v)
    plsc.subcore_barrier()

    def gather(ci, bc):
      pltpu.async_copy(xw.at[gi_v.at[pl.ds(ci * ck, ck)]], rows[bc],
                       gsem[bc])

    def load_dst(ci, bc):
      pltpu.async_copy(dp.at[wid, ci], dsts[bc], lsem[bc])

    def wait_scatter(ci, bc):
      pltpu.make_async_copy(dp.at[wid, ci], dsts[bc], lsem[bc]).wait()
      pltpu.make_async_copy(xw.at[gi_v.at[pl.ds(ci * ck, ck)]], rows[bc],
                            gsem[bc]).wait()
      pltpu.sync_copy(rows[bc], acc_s.at[dsts[bc]], add=True)

    for bc in range(2):
      load_dst(bc, bc)
      gather(bc, bc)

    def pair_body(k, _):
      ci = 2 * k
      for bc in range(2):
        wait_scatter(ci + bc, bc)
        load_dst(ci + bc + 2, bc)
        gather(ci + bc + 2, bc)
      return 0
    lax.fori_loop(0, nch // 2 - 1, pair_body, 0)
    wait_scatter(nch - 2, 0)
    wait_scatter(nch - 1, 1)

    plsc.subcore_barrier()

    for j in range(jmax):
      cid = s * jmax + j
      pltpu.sync_copy(acc_s.at[pl.ds(cid * rc, rc)],
                      acc_out.at[pl.ds(c * np_ + cid * rc, rc)])

  return body


def _sc_deg_pass(np_, e, d, nch):
  ck = 128
  rc = 80
  jmax = np_ // rc // _NS
  assert nch % 2 == 0

  mesh = plsc.VectorSubcoreMesh(core_axis_name="c", subcore_axis_name="s")

  @functools.partial(
      pl.kernel, mesh=mesh,
      out_type=jax.ShapeDtypeStruct((_NC * np_, d), jnp.float32),
      scratch_types=[
          pltpu.VMEM((ck,), jnp.int32),
          pltpu.VMEM((ck,), jnp.int32),
          pltpu.VMEM((ck, d), jnp.float32),
          pltpu.VMEM_SHARED((np_, d), jnp.float32),
          pltpu.SemaphoreType.DMA,
          pltpu.SemaphoreType.DMA,
      ])
  def body(dp, zfeat, ones, deg_out, dst0, dst1, ones_v, deg_s, l0, l1):
    c = lax.axis_index("c")
    s = lax.axis_index("s")
    wid = s * _NC + c
    dsts = (dst0, dst1)
    lsem = (l0, l1)

    for j in range(jmax):
      cid = s * jmax + j
      pltpu.sync_copy(zfeat, deg_s.at[pl.ds(cid * rc, rc)])
    pltpu.sync_copy(ones, ones_v)
    plsc.subcore_barrier()

    def load_dst(ci, bc):
      pltpu.async_copy(dp.at[wid, ci], dsts[bc], lsem[bc])

    def wait_scatter(ci, bc):
      pltpu.make_async_copy(dp.at[wid, ci], dsts[bc], lsem[bc]).wait()
      pltpu.sync_copy(ones_v, deg_s.at[dsts[bc]], add=True)

    for bc in range(2):
      load_dst(bc, bc)

    def pair_body(k, _):
      ci = 2 * k
      for bc in range(2):
        wait_scatter(ci + bc, bc)
        load_dst(ci + bc + 2, bc)
      return 0
    lax.fori_loop(0, nch // 2 - 1, pair_body, 0)
    wait_scatter(nch - 2, 0)
    wait_scatter(nch - 1, 1)

    plsc.subcore_barrier()

    for j in range(jmax):
      cid = s * jmax + j
      pltpu.sync_copy(deg_s.at[pl.ds(cid * rc, rc)],
                      deg_out.at[pl.ds(c * np_ + cid * rc, rc)])

  return body


def kernel(X, edge_index, edge_type, bases0, comb0, bases1, comb1):
  n, d = X.shape
  e = edge_type.shape[0]
  b = bases0.shape[0]
  r = comb0.shape[0]
  bn = 1000
  ck = 128
  rc = 80
  np_ = -(-n // (rc * _NS)) * (rc * _NS)
  ept = e // _NW
  nch = -(-ept // ck)
  if nch % 2:
    nch += 1
  ptile = nch * ck

  src = edge_index[0]
  dst = edge_index[1]
  e2 = e // 128
  flat_idx = _edge_index_stage(e2, n)(
      edge_type.reshape(e2, 128), src.reshape(e2, 128)).reshape(e)

  npad = ptile - ept
  tids = jnp.arange(_NW, dtype=jnp.int32)[:, None]
  js = jnp.arange(npad, dtype=jnp.int32)[None, :]
  gpad = (tids * npad + js) % (r * n)
  dpad = n + (tids * 7 + js) % (np_ - n)
  gi = jnp.concatenate([flat_idx.reshape(_NW, ept), gpad], axis=1)
  dp = jnp.concatenate([dst.reshape(_NW, ept), dpad],
                       axis=1).reshape(_NW, nch, ck)

  zfeat = jnp.zeros((rc, d), jnp.float32)
  ones = jnp.ones((ck, d), jnp.float32)

  def take(parts):
    return parts.reshape(_NC, np_, d)[:, :n]

  sc_pass = _sc_pass(np_, ptile * _NW, d, nch)
  deg = take(_sc_deg_pass(np_, ptile * _NW, d, nch)(dp, zfeat, ones))

  xw0 = _tc_stage(n, d, r, b, bn, "x", True)(X, bases0, comb0)
  acc0 = take(sc_pass(xw0.reshape(r * n, d), gi, dp, zfeat))

  xw1 = _tc_stage(n, d, r, b, bn, "norm", True)(acc0, deg, bases1, comb1)
  acc1 = take(sc_pass(xw1.reshape(r * n, d), gi, dp, zfeat))

  return _tc_stage(n, d, r, b, bn, "norm", False)(acc1, deg)


# speedup vs baseline: 28.6072x; 1.0101x over previous
import functools

import jax
import jax.numpy as jnp
from jax import lax
from jax.experimental import pallas as pl
from jax.experimental.pallas import tpu as pltpu
from jax.experimental.pallas import tpu_sc as plsc

_NC = 2
_NS = 16
_LN = 16
_NW = _NC * _NS
_DW = 16


def _tc_stage(n, d, r, b, bn, input_mode, emit_xw):
  grid = (n // bn,)

  def body(*refs):
    if input_mode == "x":
      x_ref, rest = refs[0], refs[1:]
      h = x_ref[...]
    else:
      acc_ref, deg_ref, rest = refs[0], refs[1], refs[2:]
      a = acc_ref[0] + acc_ref[1]
      dg = deg_ref[0] + deg_ref[1]
      h = a / jnp.maximum(dg, 1.0)
    if not emit_xw:
      out_ref = rest[0]
      out_ref[...] = h
      return
    bases_ref, comb_ref, out_ref = rest
    for ri in range(r):
      w = comb_ref[ri, 0] * bases_ref[0]
      for bi in range(1, b):
        w = w + comb_ref[ri, bi] * bases_ref[bi]
      out_ref[ri] = jnp.dot(h, w, preferred_element_type=jnp.float32)

  in_specs = []
  if input_mode == "x":
    in_specs.append(pl.BlockSpec((bn, d), lambda i: (i, 0)))
  else:
    in_specs.append(pl.BlockSpec((2, bn, d), lambda i: (0, i, 0)))
    in_specs.append(pl.BlockSpec((2, bn, d), lambda i: (0, i, 0)))
  if emit_xw:
    in_specs.append(pl.BlockSpec((b, d, d), lambda i: (0, 0, 0)))
    in_specs.append(pl.BlockSpec(memory_space=pltpu.SMEM))
    out_spec = pl.BlockSpec((r, bn, d), lambda i: (0, i, 0))
    out_shape = jax.ShapeDtypeStruct((r, n, d), jnp.float32)
  else:
    out_spec = pl.BlockSpec((bn, d), lambda i: (i, 0))
    out_shape = jax.ShapeDtypeStruct((n, d), jnp.float32)

  return pl.pallas_call(
      body, grid=grid, in_specs=in_specs, out_specs=out_spec,
      out_shape=out_shape)


def _edge_index_stage(e2, n):
  def body(et_ref, src_ref, out_ref):
    out_ref[...] = et_ref[...] * n + src_ref[...]

  return pl.pallas_call(
      body,
      out_shape=jax.ShapeDtypeStruct((e2, 128), jnp.int32))


def _sc_pass(np_, e, d, nch, ck):
  rc = 80
  jmax = np_ // rc // _NS
  assert e == nch * ck * _NW and np_ % (rc * _NS) == 0
  assert nch % 4 == 0 and nch >= 8 and ck <= 128

  mesh = plsc.VectorSubcoreMesh(core_axis_name="c", subcore_axis_name="s")

  @functools.partial(
      pl.kernel, mesh=mesh,
      out_type=jax.ShapeDtypeStruct((_NC * np_, d), jnp.float32),
      scratch_types=(
          [pltpu.VMEM((nch * ck,), jnp.int32)]
          + [pltpu.VMEM((ck,), jnp.int32)] * 4
          + [pltpu.VMEM((ck, d), jnp.float32)] * 4
          + [pltpu.VMEM_SHARED((np_, d), jnp.float32)]
          + [pltpu.SemaphoreType.DMA] * 12))
  def body(xw, gi, dp, zfeat, acc_out, gi_v, d0, d1, d2, d3,
           r0, r1, r2, r3, acc_s, *sems):
    c = lax.axis_index("c")
    s = lax.axis_index("s")
    wid = s * _NC + c
    dsts = (d0, d1, d2, d3)
    rows = (r0, r1, r2, r3)
    gsem, lsem, ssem = sems[0:4], sems[4:8], sems[8:12]

    for j in range(jmax):
      cid = s * jmax + j
      pltpu.sync_copy(zfeat, acc_s.at[pl.ds(cid * rc, rc)])
    pltpu.sync_copy(gi.at[wid], gi_v)
    plsc.subcore_barrier()

    def issue(ci, bc):
      pltpu.async_copy(dp.at[wid, ci], dsts[bc], lsem[bc])
      pltpu.async_copy(xw.at[gi_v.at[pl.ds(ci * ck, ck)]], rows[bc],
                       gsem[bc])

    def scatter(ci, bc):
      pltpu.make_async_copy(dp.at[wid, ci], dsts[bc], lsem[bc]).wait()
      pltpu.make_async_copy(xw.at[gi_v.at[pl.ds(ci * ck, ck)]], rows[bc],
                            gsem[bc]).wait()
      pltpu.async_copy(rows[bc], acc_s.at[dsts[bc]], ssem[bc], add=True)

    def wait_scatter(bc):
      pltpu.make_async_copy(rows[bc], acc_s.at[dsts[bc]], ssem[bc]).wait()

    issue(0, 0)
    issue(1, 1)
    for ci in range(2):
      issue(ci + 2, ci + 2)
      scatter(ci, ci)

    def quad_body(k, _):
      ci = 4 * k + 2
      for i, bc in enumerate((2, 3, 0, 1)):
        wait_scatter((bc + 2) % 4)
        issue(ci + i + 2, (bc + 2) % 4)
        scatter(ci + i, bc)
      return 0
    lax.fori_loop(0, nch // 4 - 1, quad_body, 0)
    for i, bc in enumerate((2, 3)):
      wait_scatter(bc - 2)
      scatter(nch - 2 + i, bc)
    wait_scatter(2)
    wait_scatter(3)

    plsc.subcore_barrier()

    for j in range(jmax):
      cid = s * jmax + j
      pltpu.sync_copy(acc_s.at[pl.ds(cid * rc, rc)],
                      acc_out.at[pl.ds(c * np_ + cid * rc, rc)])

  return body


def _sc_deg_pass(np_, e, d, nch):
  ck = 128
  rc = 80
  jmax = np_ // rc // _NS
  assert nch % 4 == 0 and nch >= 8

  mesh = plsc.VectorSubcoreMesh(core_axis_name="c", subcore_axis_name="s")

  @functools.partial(
      pl.kernel, mesh=mesh,
      out_type=jax.ShapeDtypeStruct((_NC * np_, d), jnp.float32),
      scratch_types=(
          [pltpu.VMEM((ck,), jnp.int32)] * 4
          + [pltpu.VMEM((ck, d), jnp.float32)]
          + [pltpu.VMEM_SHARED((np_, d), jnp.float32)]
          + [pltpu.SemaphoreType.DMA] * 8))
  def body(dp, zfeat, ones, deg_out, d0, d1, d2, d3, ones_v, deg_s, *sems):
    c = lax.axis_index("c")
    s = lax.axis_index("s")
    wid = s * _NC + c
    dsts = (d0, d1, d2, d3)
    lsem, ssem = sems[0:4], sems[4:8]

    for j in range(jmax):
      cid = s * jmax + j
      pltpu.sync_copy(zfeat, deg_s.at[pl.ds(cid * rc, rc)])
    pltpu.sync_copy(ones, ones_v)
    plsc.subcore_barrier()

    def issue(ci, bc):
      pltpu.async_copy(dp.at[wid, ci], dsts[bc], lsem[bc])

    def scatter(ci, bc):
      pltpu.make_async_copy(dp.at[wid, ci], dsts[bc], lsem[bc]).wait()
      pltpu.async_copy(ones_v, deg_s.at[dsts[bc]], ssem[bc], add=True)

    def wait_scatter(bc):
      pltpu.make_async_copy(ones_v, deg_s.at[dsts[bc]], ssem[bc]).wait()

    issue(0, 0)
    issue(1, 1)
    for ci in range(2):
      issue(ci + 2, ci + 2)
      scatter(ci, ci)

    def quad_body(k, _):
      ci = 4 * k + 2
      for i, bc in enumerate((2, 3, 0, 1)):
        wait_scatter((bc + 2) % 4)
        issue(ci + i + 2, (bc + 2) % 4)
        scatter(ci + i, bc)
      return 0
    lax.fori_loop(0, nch // 4 - 1, quad_body, 0)
    for i, bc in enumerate((2, 3)):
      wait_scatter(bc - 2)
      scatter(nch - 2 + i, bc)
    wait_scatter(2)
    wait_scatter(3)

    plsc.subcore_barrier()

    for j in range(jmax):
      cid = s * jmax + j
      pltpu.sync_copy(deg_s.at[pl.ds(cid * rc, rc)],
                      deg_out.at[pl.ds(c * np_ + cid * rc, rc)])

  return body


def kernel(X, edge_index, edge_type, bases0, comb0, bases1, comb1):
  n, d = X.shape
  e = edge_type.shape[0]
  b = bases0.shape[0]
  r = comb0.shape[0]
  bn = 1000
  ck = 128
  rc = 80
  np_ = -(-n // (rc * _NS)) * (rc * _NS)
  ept = e // _NW
  nch = -(-ept // ck)
  if nch % 2:
    nch += 1
  ptile = nch * ck

  src = edge_index[0]
  dst = edge_index[1]
  e2 = e // 128
  flat_idx = _edge_index_stage(e2, n)(
      edge_type.reshape(e2, 128), src.reshape(e2, 128)).reshape(e)

  npad = ptile - ept
  tids = jnp.arange(_NW, dtype=jnp.int32)[:, None]
  js = jnp.arange(npad, dtype=jnp.int32)[None, :]
  gpad = (tids * npad + js) % (r * n)
  dpad = n + (tids * 7 + js) % (np_ - n)
  gi = jnp.concatenate([flat_idx.reshape(_NW, ept), gpad], axis=1)
  dp = jnp.concatenate([dst.reshape(_NW, ept), dpad],
                       axis=1).reshape(_NW, nch, ck)

  zfeat = jnp.zeros((rc, d), jnp.float32)
  ones = jnp.ones((ck, d), jnp.float32)

  def take(parts):
    return parts.reshape(_NC, np_, d)[:, :n]

  cka = 64
  ncha = ptile // cka
  dpa = dp.reshape(_NW, ncha, cka)
  sc_pass = _sc_pass(np_, ptile * _NW, d, ncha, cka)
  deg = take(_sc_deg_pass(np_, ptile * _NW, d, nch)(dp, zfeat, ones))

  xw0 = _tc_stage(n, d, r, b, bn, "x", True)(X, bases0, comb0)
  acc0 = take(sc_pass(xw0.reshape(r * n, d), gi, dpa, zfeat))

  xw1 = _tc_stage(n, d, r, b, bn, "norm", True)(acc0, deg, bases1, comb1)
  acc1 = take(sc_pass(xw1.reshape(r * n, d), gi, dpa, zfeat))

  return _tc_stage(n, d, r, b, bn, "norm", False)(acc1, deg)


# speedup vs baseline: 32.0763x; 1.1213x over previous
import functools

import jax
import jax.numpy as jnp
from jax import lax
from jax.experimental import pallas as pl
from jax.experimental.pallas import tpu as pltpu
from jax.experimental.pallas import tpu_sc as plsc

_NC = 2
_NS = 16
_LN = 16
_NW = _NC * _NS
_DW = 16


def _tc_stage(n, d, r, b, bn, input_mode, emit_xw):
  grid = (n // bn,)

  def body(*refs):
    if input_mode == "x":
      x_ref, rest = refs[0], refs[1:]
      h = x_ref[...]
    else:
      acc_ref, deg0_ref, deg1_ref, rest = refs[0], refs[1], refs[2], refs[3:]
      a = acc_ref[0] + acc_ref[1]
      dg = deg0_ref[...] + deg1_ref[...]
      h = a / jnp.maximum(dg, 1.0)
    if not emit_xw:
      out_ref = rest[0]
      out_ref[...] = h
      return
    bases_ref, comb_ref, out_ref = rest
    for ri in range(r):
      w = comb_ref[ri, 0] * bases_ref[0]
      for bi in range(1, b):
        w = w + comb_ref[ri, bi] * bases_ref[bi]
      out_ref[ri] = jnp.dot(h, w, preferred_element_type=jnp.float32)

  in_specs = []
  if input_mode == "x":
    in_specs.append(pl.BlockSpec((bn, d), lambda i: (i, 0)))
  else:
    in_specs.append(pl.BlockSpec((2, bn, d), lambda i: (0, i, 0)))
    in_specs.append(pl.BlockSpec((bn, 1), lambda i: (i, 0)))
    in_specs.append(pl.BlockSpec((bn, 1), lambda i: (i, 0)))
  if emit_xw:
    in_specs.append(pl.BlockSpec((b, d, d), lambda i: (0, 0, 0)))
    in_specs.append(pl.BlockSpec(memory_space=pltpu.SMEM))
    out_spec = pl.BlockSpec((r, bn, d), lambda i: (0, i, 0))
    out_shape = jax.ShapeDtypeStruct((r, n, d), jnp.float32)
  else:
    out_spec = pl.BlockSpec((bn, d), lambda i: (i, 0))
    out_shape = jax.ShapeDtypeStruct((n, d), jnp.float32)

  return pl.pallas_call(
      body, grid=grid, in_specs=in_specs, out_specs=out_spec,
      out_shape=out_shape)


def _edge_index_stage(e2, n):
  def body(et_ref, src_ref, out_ref):
    out_ref[...] = et_ref[...] * n + src_ref[...]

  return pl.pallas_call(
      body,
      out_shape=jax.ShapeDtypeStruct((e2, 128), jnp.int32))


def _sc_pass(np_, e, d, nch, ck):
  rc = 80
  jmax = np_ // rc // _NS
  assert e == nch * ck * _NW and np_ % (rc * _NS) == 0
  assert nch % 4 == 0 and nch >= 8 and ck <= 128

  mesh = plsc.VectorSubcoreMesh(core_axis_name="c", subcore_axis_name="s")

  @functools.partial(
      pl.kernel, mesh=mesh,
      out_type=jax.ShapeDtypeStruct((_NC * np_, d), jnp.float32),
      scratch_types=(
          [pltpu.VMEM((nch * ck,), jnp.int32)]
          + [pltpu.VMEM((ck,), jnp.int32)] * 4
          + [pltpu.VMEM((ck, d), jnp.float32)] * 4
          + [pltpu.VMEM_SHARED((np_, d), jnp.float32)]
          + [pltpu.SemaphoreType.DMA] * 12))
  def body(xw, gi, dp, zfeat, acc_out, gi_v, d0, d1, d2, d3,
           r0, r1, r2, r3, acc_s, *sems):
    c = lax.axis_index("c")
    s = lax.axis_index("s")
    wid = s * _NC + c
    dsts = (d0, d1, d2, d3)
    rows = (r0, r1, r2, r3)
    gsem, lsem, ssem = sems[0:4], sems[4:8], sems[8:12]

    for j in range(jmax):
      cid = s * jmax + j
      pltpu.sync_copy(zfeat, acc_s.at[pl.ds(cid * rc, rc)])
    pltpu.sync_copy(gi.at[wid], gi_v)
    plsc.subcore_barrier()

    def issue(ci, bc):
      pltpu.async_copy(dp.at[wid, ci], dsts[bc], lsem[bc])
      pltpu.async_copy(xw.at[gi_v.at[pl.ds(ci * ck, ck)]], rows[bc],
                       gsem[bc])

    def scatter(ci, bc):
      pltpu.make_async_copy(dp.at[wid, ci], dsts[bc], lsem[bc]).wait()
      pltpu.make_async_copy(xw.at[gi_v.at[pl.ds(ci * ck, ck)]], rows[bc],
                            gsem[bc]).wait()
      pltpu.async_copy(rows[bc], acc_s.at[dsts[bc]], ssem[bc], add=True)

    def wait_scatter(bc):
      pltpu.make_async_copy(rows[bc], acc_s.at[dsts[bc]], ssem[bc]).wait()

    issue(0, 0)
    issue(1, 1)
    for ci in range(2):
      issue(ci + 2, ci + 2)
      scatter(ci, ci)

    def quad_body(k, _):
      ci = 4 * k + 2
      for i, bc in enumerate((2, 3, 0, 1)):
        wait_scatter((bc + 2) % 4)
        issue(ci + i + 2, (bc + 2) % 4)
        scatter(ci + i, bc)
      return 0
    lax.fori_loop(0, nch // 4 - 1, quad_body, 0)
    for i, bc in enumerate((2, 3)):
      wait_scatter(bc - 2)
      scatter(nch - 2 + i, bc)
    wait_scatter(2)
    wait_scatter(3)

    plsc.subcore_barrier()

    for j in range(jmax):
      cid = s * jmax + j
      pltpu.sync_copy(acc_s.at[pl.ds(cid * rc, rc)],
                      acc_out.at[pl.ds(c * np_ + cid * rc, rc)])

  return body


def _sc_deg_pass(np_, d, nch):
  ck = 128
  rsl = np_ // _NS
  assert nch % 2 == 0 and rsl % _LN == 0

  mesh = plsc.VectorSubcoreMesh(core_axis_name="c", subcore_axis_name="s")

  @functools.partial(
      pl.kernel, mesh=mesh,
      out_type=jax.ShapeDtypeStruct((_NC * np_,), jnp.float32),
      compiler_params=pltpu.CompilerParams(needs_layout_passes=False),
      scratch_types=(
          [pltpu.VMEM((ck,), jnp.int32)] * 2
          + [pltpu.VMEM((np_,), jnp.float32)]
          + [pltpu.VMEM((_NS, rsl), jnp.float32)]
          + [pltpu.VMEM((rsl,), jnp.float32)]
          + [pltpu.VMEM_SHARED((_NS, np_), jnp.float32)]
          + [pltpu.SemaphoreType.DMA] * 2))
  def body(dp, zhist, deg_out, d0, d1, hist_v, hsl_v, red_v, stage_s,
           l0, l1):
    c = lax.axis_index("c")
    s = lax.axis_index("s")
    wid = s * _NC + c
    dsts = (d0, d1)
    lsem = (l0, l1)
    onesv = jnp.ones((_LN,), jnp.float32)

    pltpu.sync_copy(zhist, hist_v)

    def load(ci, bc):
      pltpu.async_copy(dp.at[wid, ci], dsts[bc], lsem[bc])

    def count(ci, bc):
      pltpu.make_async_copy(dp.at[wid, ci], dsts[bc], lsem[bc]).wait()
      for j in range(ck // _LN):
        idx16 = dsts[bc][pl.ds(j * _LN, _LN)]
        plsc.addupdate_scatter(hist_v, [idx16], onesv)

    load(0, 0)
    load(1, 1)

    def pair_body(k, _):
      ci = 2 * k
      for bc in range(2):
        count(ci + bc, bc)
        load(ci + bc + 2, bc)
      return 0
    lax.fori_loop(0, nch // 2 - 1, pair_body, 0)
    count(nch - 2, 0)
    count(nch - 1, 1)

    pltpu.sync_copy(hist_v, stage_s.at[s])
    plsc.subcore_barrier()
    for t in range(_NS):
      pltpu.sync_copy(stage_s.at[t, pl.ds(s * rsl, rsl)], hsl_v.at[t])
    for j in range(rsl // _LN):
      sl = pl.ds(j * _LN, _LN)
      acc16 = hsl_v[0, sl]
      for t in range(1, _NS):
        acc16 = acc16 + hsl_v[t, sl]
      red_v[sl] = acc16
    pltpu.sync_copy(red_v, deg_out.at[pl.ds(c * np_ + s * rsl, rsl)])

  return body


def kernel(X, edge_index, edge_type, bases0, comb0, bases1, comb1):
  n, d = X.shape
  e = edge_type.shape[0]
  b = bases0.shape[0]
  r = comb0.shape[0]
  bn = 1000
  ck = 128
  rc = 80
  np_ = -(-n // (rc * _NS)) * (rc * _NS)
  ept = e // _NW
  nch = -(-ept // ck)
  if nch % 2:
    nch += 1
  ptile = nch * ck

  src = edge_index[0]
  dst = edge_index[1]
  e2 = e // 128
  flat_idx = _edge_index_stage(e2, n)(
      edge_type.reshape(e2, 128), src.reshape(e2, 128)).reshape(e)

  npad = ptile - ept
  tids = jnp.arange(_NW, dtype=jnp.int32)[:, None]
  js = jnp.arange(npad, dtype=jnp.int32)[None, :]
  gpad = (tids * npad + js) % (r * n)
  dpad = n + (tids * 7 + js) % (np_ - n)
  gi = jnp.concatenate([flat_idx.reshape(_NW, ept), gpad], axis=1)
  dp = jnp.concatenate([dst.reshape(_NW, ept), dpad],
                       axis=1).reshape(_NW, nch, ck)

  zfeat = jnp.zeros((rc, d), jnp.float32)
  ones = jnp.ones((ck, d), jnp.float32)

  def take(parts):
    return parts.reshape(_NC, np_, d)[:, :n]

  cka = 64
  ncha = ptile // cka
  dpa = dp.reshape(_NW, ncha, cka)
  sc_pass = _sc_pass(np_, ptile * _NW, d, ncha, cka)
  zhist = jnp.zeros((np_,), jnp.float32)
  deg1d = _sc_deg_pass(np_, d, nch)(dp, zhist)
  d0 = deg1d[:np_][:n].reshape(n, 1)
  d1 = deg1d[np_:][:n].reshape(n, 1)

  xw0 = _tc_stage(n, d, r, b, bn, "x", True)(X, bases0, comb0)
  acc0 = take(sc_pass(xw0.reshape(r * n, d), gi, dpa, zfeat))

  xw1 = _tc_stage(n, d, r, b, bn, "norm", True)(acc0, d0, d1, bases1, comb1)
  acc1 = take(sc_pass(xw1.reshape(r * n, d), gi, dpa, zfeat))

  return _tc_stage(n, d, r, b, bn, "norm", False)(acc1, d0, d1)


# speedup vs baseline: 33.7640x; 1.0526x over previous
import functools

import jax
import jax.numpy as jnp
from jax import lax
from jax.experimental import pallas as pl
from jax.experimental.pallas import tpu as pltpu
from jax.experimental.pallas import tpu_sc as plsc

_NC = 2
_NS = 16
_LN = 16
_NW = _NC * _NS
_DW = 16


def _tc_stage(n, d, r, b, bn, input_mode, emit_xw):
  grid = (n // bn,)

  def body(*refs):
    if input_mode == "x":
      x_ref, rest = refs[0], refs[1:]
      h = x_ref[...]
    else:
      acc_ref, deg0_ref, deg1_ref, rest = refs[0], refs[1], refs[2], refs[3:]
      a = acc_ref[0] + acc_ref[1]
      dg = deg0_ref[...] + deg1_ref[...]
      h = a / jnp.maximum(dg, 1.0)
    if not emit_xw:
      out_ref = rest[0]
      out_ref[...] = h
      return
    bases_ref, comb_ref, out_ref = rest
    for ri in range(r):
      w = comb_ref[ri, 0] * bases_ref[0]
      for bi in range(1, b):
        w = w + comb_ref[ri, bi] * bases_ref[bi]
      out_ref[ri] = jnp.dot(h, w, preferred_element_type=jnp.float32)

  in_specs = []
  if input_mode == "x":
    in_specs.append(pl.BlockSpec((bn, d), lambda i: (i, 0)))
  else:
    in_specs.append(pl.BlockSpec((2, bn, d), lambda i: (0, i, 0)))
    in_specs.append(pl.BlockSpec((bn, 1), lambda i: (i, 0)))
    in_specs.append(pl.BlockSpec((bn, 1), lambda i: (i, 0)))
  if emit_xw:
    in_specs.append(pl.BlockSpec((b, d, d), lambda i: (0, 0, 0)))
    in_specs.append(pl.BlockSpec(memory_space=pltpu.SMEM))
    out_spec = pl.BlockSpec((r, bn, d), lambda i: (0, i, 0))
    out_shape = jax.ShapeDtypeStruct((r, n, d), jnp.float32)
  else:
    out_spec = pl.BlockSpec((bn, d), lambda i: (i, 0))
    out_shape = jax.ShapeDtypeStruct((n, d), jnp.float32)

  return pl.pallas_call(
      body, grid=grid, in_specs=in_specs, out_specs=out_spec,
      out_shape=out_shape)


def _edge_index_stage(e2, n):
  def body(et_ref, src_ref, out_ref):
    out_ref[...] = et_ref[...] * n + src_ref[...]

  return pl.pallas_call(
      body,
      out_shape=jax.ShapeDtypeStruct((e2, 128), jnp.int32))


def _sc_pass(np_, e, d, nch, ck):
  rc = 80
  jmax = np_ // rc // _NS
  assert e == nch * ck * _NW and np_ % (rc * _NS) == 0
  assert nch % 4 == 0 and nch >= 8 and ck <= 128

  mesh = plsc.VectorSubcoreMesh(core_axis_name="c", subcore_axis_name="s")

  @functools.partial(
      pl.kernel, mesh=mesh,
      out_type=jax.ShapeDtypeStruct((_NC * np_, d), jnp.float32),
      scratch_types=(
          [pltpu.VMEM((nch * ck,), jnp.int32)]
          + [pltpu.VMEM((ck,), jnp.int32)] * 3
          + [pltpu.VMEM((ck, d), jnp.float32)] * 3
          + [pltpu.VMEM_SHARED((np_, d), jnp.float32)]
          + [pltpu.SemaphoreType.DMA] * 9))
  def body(xw, gi, dp, zfeat, acc_out, gi_v, d0, d1, d2,
           r0, r1, r2, acc_s, *sems):
    c = lax.axis_index("c")
    s = lax.axis_index("s")
    wid = s * _NC + c
    dsts = (d0, d1, d2)
    rows = (r0, r1, r2)
    gsem, lsem, ssem = sems[0:3], sems[3:6], sems[6:9]

    for j in range(jmax):
      cid = s * jmax + j
      pltpu.sync_copy(zfeat, acc_s.at[pl.ds(cid * rc, rc)])
    pltpu.sync_copy(gi.at[wid], gi_v)
    plsc.subcore_barrier()

    def issue(ci, bc):
      pltpu.async_copy(dp.at[wid, ci], dsts[bc], lsem[bc])
      pltpu.async_copy(xw.at[gi_v.at[pl.ds(ci * ck, ck)]], rows[bc],
                       gsem[bc])

    def scatter(ci, bc):
      pltpu.make_async_copy(dp.at[wid, ci], dsts[bc], lsem[bc]).wait()
      pltpu.make_async_copy(xw.at[gi_v.at[pl.ds(ci * ck, ck)]], rows[bc],
                            gsem[bc]).wait()
      pltpu.async_copy(rows[bc], acc_s.at[dsts[bc]], ssem[bc], add=True)

    def wait_scatter(bc):
      pltpu.make_async_copy(rows[bc], acc_s.at[dsts[bc]], ssem[bc]).wait()

    def step(ci, bc, first=False, last=False):
      pltpu.make_async_copy(dp.at[wid, ci], dsts[bc], lsem[bc]).wait()
      pltpu.make_async_copy(xw.at[gi_v.at[pl.ds(ci * ck, ck)]], rows[bc],
                            gsem[bc]).wait()
      if not first:
        wait_scatter((bc + 2) % 3)
      pltpu.async_copy(rows[bc], acc_s.at[dsts[bc]], ssem[bc], add=True)
      if not last:
        issue(ci + 2, (bc + 2) % 3)

    issue(0, 0)
    issue(1, 1)
    step(0, 0, first=True)

    m = (nch - 3) // 3

    def tri_body(k, _):
      ci = 3 * k + 1
      for i in range(3):
        step(ci + i, (1 + i) % 3)
      return 0
    lax.fori_loop(0, m, tri_body, 0)
    for ci in range(1 + 3 * m, nch):
      step(ci, ci % 3, last=(ci + 2 > nch - 1))
    wait_scatter((nch - 1) % 3)

    plsc.subcore_barrier()

    for j in range(jmax):
      cid = s * jmax + j
      pltpu.sync_copy(acc_s.at[pl.ds(cid * rc, rc)],
                      acc_out.at[pl.ds(c * np_ + cid * rc, rc)])

  return body


def _sc_deg_pass(np_, d, nch):
  ck = 128
  rsl = np_ // _NS
  assert nch % 2 == 0 and rsl % _LN == 0

  mesh = plsc.VectorSubcoreMesh(core_axis_name="c", subcore_axis_name="s")

  @functools.partial(
      pl.kernel, mesh=mesh,
      out_type=jax.ShapeDtypeStruct((_NC * np_,), jnp.float32),
      compiler_params=pltpu.CompilerParams(needs_layout_passes=False),
      scratch_types=(
          [pltpu.VMEM((ck,), jnp.int32)] * 2
          + [pltpu.VMEM((np_,), jnp.float32)]
          + [pltpu.VMEM((_NS, rsl), jnp.float32)]
          + [pltpu.VMEM((rsl,), jnp.float32)]
          + [pltpu.VMEM_SHARED((_NS, np_), jnp.float32)]
          + [pltpu.SemaphoreType.DMA] * 2))
  def body(dp, zhist, deg_out, d0, d1, hist_v, hsl_v, red_v, stage_s,
           l0, l1):
    c = lax.axis_index("c")
    s = lax.axis_index("s")
    wid = s * _NC + c
    dsts = (d0, d1)
    lsem = (l0, l1)
    onesv = jnp.ones((_LN,), jnp.float32)

    pltpu.sync_copy(zhist, hist_v)

    def load(ci, bc):
      pltpu.async_copy(dp.at[wid, ci], dsts[bc], lsem[bc])

    def count(ci, bc):
      pltpu.make_async_copy(dp.at[wid, ci], dsts[bc], lsem[bc]).wait()
      for j in range(ck // _LN):
        idx16 = dsts[bc][pl.ds(j * _LN, _LN)]
        plsc.addupdate_scatter(hist_v, [idx16], onesv)

    load(0, 0)
    load(1, 1)

    def pair_body(k, _):
      ci = 2 * k
      for bc in range(2):
        count(ci + bc, bc)
        load(ci + bc + 2, bc)
      return 0
    lax.fori_loop(0, nch // 2 - 1, pair_body, 0)
    count(nch - 2, 0)
    count(nch - 1, 1)

    pltpu.sync_copy(hist_v, stage_s.at[s])
    plsc.subcore_barrier()
    for t in range(_NS):
      pltpu.sync_copy(stage_s.at[t, pl.ds(s * rsl, rsl)], hsl_v.at[t])
    for j in range(rsl // _LN):
      sl = pl.ds(j * _LN, _LN)
      acc16 = hsl_v[0, sl]
      for t in range(1, _NS):
        acc16 = acc16 + hsl_v[t, sl]
      red_v[sl] = acc16
    pltpu.sync_copy(red_v, deg_out.at[pl.ds(c * np_ + s * rsl, rsl)])

  return body


def kernel(X, edge_index, edge_type, bases0, comb0, bases1, comb1):
  n, d = X.shape
  e = edge_type.shape[0]
  b = bases0.shape[0]
  r = comb0.shape[0]
  bn = 2000
  ck = 128
  rc = 80
  np_ = -(-n // (rc * _NS)) * (rc * _NS)
  ept = e // _NW
  nch = -(-ept // ck)
  if nch % 2:
    nch += 1
  ptile = nch * ck

  src = edge_index[0]
  dst = edge_index[1]
  e2 = e // 128
  flat_idx = _edge_index_stage(e2, n)(
      edge_type.reshape(e2, 128), src.reshape(e2, 128)).reshape(e)

  npad = ptile - ept
  tids = jnp.arange(_NW, dtype=jnp.int32)[:, None]
  js = jnp.arange(npad, dtype=jnp.int32)[None, :]
  gpad = (tids * npad + js) % (r * n)
  dpad = n + (tids * 7 + js) % (np_ - n)
  gi = jnp.concatenate([flat_idx.reshape(_NW, ept), gpad], axis=1)
  dp = jnp.concatenate([dst.reshape(_NW, ept), dpad],
                       axis=1).reshape(_NW, nch, ck)

  zfeat = jnp.zeros((rc, d), jnp.float32)

  def take(parts):
    return parts.reshape(_NC, np_, d)[:, :n]

  cka = 80
  ncha = ptile // cka
  dpa = dp.reshape(_NW, ncha, cka)
  sc_pass = _sc_pass(np_, ptile * _NW, d, ncha, cka)
  zhist = jnp.zeros((np_,), jnp.float32)
  deg1d = _sc_deg_pass(np_, d, nch)(dp, zhist)
  d0 = deg1d[:np_][:n].reshape(n, 1)
  d1 = deg1d[np_:][:n].reshape(n, 1)

  xw0 = _tc_stage(n, d, r, b, bn, "x", True)(X, bases0, comb0)
  acc0 = take(sc_pass(xw0.reshape(r * n, d), gi, dpa, zfeat))

  xw1 = _tc_stage(n, d, r, b, bn, "norm", True)(acc0, d0, d1, bases1, comb1)
  acc1 = take(sc_pass(xw1.reshape(r * n, d), gi, dpa, zfeat))

  return _tc_stage(n, d, r, b, bn, "norm", False)(acc1, d0, d1)
